# baseline (speedup 1.0000x reference)
import functools

import jax
import jax.numpy as jnp
from jax import lax
from jax.experimental import pallas as pl
from jax.experimental.pallas import tpu as pltpu, tpu_sc as plsc

N = 10000
E = 320000
DF = 128
H = 64
T = 8
G = 128

NC = 2
NS = 16
NW = NC * NS
CH = 128
NBUF = 2
K = -(-E // (NW * CH * NBUF)) * NBUF
E_PAD = NW * K * CH
NPAD = -(-N // (NS * 8)) * NS * 8
RPW = NPAD // NS


def _bn_relu(h, g, b):
    m = jnp.mean(h, axis=0, keepdims=True)
    v = jnp.mean((h - m) * (h - m), axis=0, keepdims=True)
    return jax.nn.relu((h - m) / jnp.sqrt(v + 1e-5) * g + b)


def _onehot_t(batch_ref):
    rows = lax.broadcasted_iota(jnp.int32, (G, N), 0)
    return (rows == batch_ref[...]).astype(jnp.float32)


def _mlp_pool_body(bias_per_node, x_ref, batch_ref, w1, b1, g1, be1,
                   w2, b2, g2, be2, lw, lb, part_in, h_out, part_out):
    h = jnp.dot(x_ref[...], w1[...], preferred_element_type=jnp.float32)
    h = _bn_relu(h + b1[...], g1[...], be1[...])
    h = jnp.dot(h, w2[...], preferred_element_type=jnp.float32)
    h = _bn_relu(h + b2[...], g2[...], be2[...])
    h_out[...] = h
    oh = _onehot_t(batch_ref)
    pool = jnp.dot(oh, h, preferred_element_type=jnp.float32)
    if bias_per_node:
        bterm = jnp.sum(oh, axis=1, keepdims=True) * lb[...]
    else:
        bterm = lb[...]
    part_out[...] = (part_in[...]
                     + jnp.dot(pool, lw[...], preferred_element_type=jnp.float32)
                     + bterm)


def _gin_input_body(h_ref, a0_ref, a1_ref, eps_ref, out_ref):
    out_ref[...] = ((1.0 + eps_ref[0, 0]) * h_ref[...]
                    + a0_ref[0:N, :] + a1_ref[0:N, :])


_out_shapes = (jax.ShapeDtypeStruct((N, H), jnp.float32),
               jax.ShapeDtypeStruct((G, T), jnp.float32))
_mlp_pool0 = pl.pallas_call(
    functools.partial(_mlp_pool_body, True), out_shape=_out_shapes)
_mlp_pool = pl.pallas_call(
    functools.partial(_mlp_pool_body, False), out_shape=_out_shapes)

_gin_input = pl.pallas_call(
    _gin_input_body,
    out_shape=jax.ShapeDtypeStruct((N, H), jnp.float32),
)


def _sc_agg_body(h_hbm, srcs_hbm, dsts_hbm, out_hbm,
                 src_v, dst_v, rows, gsems, ssems, obuf, acc):
    cid = lax.axis_index("c")
    sid = lax.axis_index("s")
    wid = sid * NC + cid

    def _z(j, _):
        obuf[j // (H // 16), pl.ds((j % (H // 16)) * 16, 16)] = (
            jnp.zeros((16,), jnp.float32))
        return 0
    lax.fori_loop(0, RPW * (H // 16), _z, 0)
    pltpu.sync_copy(obuf, acc.at[pl.ds(sid * RPW, RPW)])
    plsc.subcore_barrier()

    pltpu.sync_copy(srcs_hbm.at[wid], src_v)
    pltpu.sync_copy(dsts_hbm.at[wid], dst_v)

    for b in range(NBUF):
        pltpu.async_copy(h_hbm.at[src_v.at[b]], rows[b], gsems[b])

    def _body(i, _):
        c = i * NBUF
        for b in range(NBUF):
            pltpu.make_async_copy(h_hbm.at[src_v.at[c + b]],
                                  rows[b], gsems[b]).wait()
            pltpu.async_copy(rows[b], acc.at[dst_v.at[c + b]], ssems[b],
                             add=True)
        for b in range(NBUF):
            pltpu.make_async_copy(rows[b],
                                  acc.at[dst_v.at[c + b]], ssems[b]).wait()

            @pl.when(i < K // NBUF - 1)
            def _():
                pltpu.async_copy(h_hbm.at[src_v.at[c + NBUF + b]],
                                 rows[b], gsems[b])
        return 0
    lax.fori_loop(0, K // NBUF, _body, 0)
    plsc.subcore_barrier()

    pltpu.sync_copy(acc.at[pl.ds(sid * RPW, RPW)], obuf)
    pltpu.sync_copy(obuf, out_hbm.at[cid, pl.ds(sid * RPW, RPW)])


@functools.lru_cache(maxsize=None)
def _make_sc_agg():
    return pl.kernel(
        _sc_agg_body,
        out_type=jax.ShapeDtypeStruct((NC, NPAD, H), jnp.float32),
        mesh=plsc.VectorSubcoreMesh(core_axis_name="c", subcore_axis_name="s",
                                    num_cores=NC, num_subcores=NS),
        scratch_types=[
            pltpu.VMEM((K, CH), jnp.int32),
            pltpu.VMEM((K, CH), jnp.int32),
            [pltpu.VMEM((CH, H), jnp.float32) for _ in range(NBUF)],
            [pltpu.SemaphoreType.DMA for _ in range(NBUF)],
            [pltpu.SemaphoreType.DMA for _ in range(NBUF)],
            pltpu.VMEM((RPW, H), jnp.float32),
            pltpu.VMEM_SHARED((NPAD, H), jnp.float32),
        ],
        compiler_params=pltpu.CompilerParams(use_tc_tiling_on_sc=False),
    )


def _sc_agg(h, src, dst):
    return _make_sc_agg()(h, src, dst)


def kernel(x, edge_index, batch, fh_W1, fh_b1, fh_g1, fh_be1, fh_W2, fh_b2,
           fh_g2, fh_be2, c0_W1, c0_b1, c0_g1, c0_be1, c0_W2, c0_b2, c0_g2,
           c0_be2, c0_eps, c1_W1, c1_b1, c1_g1, c1_be1, c1_W2, c1_b2, c1_g2,
           c1_be2, c1_eps, lin0_W, lin0_b, lin1_W, lin1_b, lin2_W, lin2_b):
    r = lambda a: a.reshape(1, -1)
    batch_r = batch.reshape(1, N)

    src = jnp.concatenate(
        [edge_index[0], jnp.zeros((E_PAD - E,), jnp.int32)]).reshape(NW, K, CH)
    dst = jnp.concatenate(
        [edge_index[1], jnp.full((E_PAD - E,), N, jnp.int32)]).reshape(NW, K, CH)

    zpart = jnp.zeros((G, T), jnp.float32)

    h0, part0 = _mlp_pool0(x, batch_r, fh_W1, r(fh_b1), r(fh_g1), r(fh_be1),
                           fh_W2, r(fh_b2), r(fh_g2), r(fh_be2),
                           lin0_W, r(lin0_b), zpart)

    agg = _sc_agg(h0, src, dst)
    x1 = _gin_input(h0, agg[0], agg[1], c0_eps.reshape(1, 1))
    h1, part1 = _mlp_pool(x1, batch_r, c0_W1, r(c0_b1), r(c0_g1), r(c0_be1),
                          c0_W2, r(c0_b2), r(c0_g2), r(c0_be2),
                          lin1_W, r(lin1_b), part0)

    agg = _sc_agg(h1, src, dst)
    x2 = _gin_input(h1, agg[0], agg[1], c1_eps.reshape(1, 1))
    _, part2 = _mlp_pool(x2, batch_r, c1_W1, r(c1_b1), r(c1_g1), r(c1_be1),
                         c1_W2, r(c1_b2), r(c1_g2), r(c1_be2),
                         lin2_W, r(lin2_b), part1)
    return part2


# speedup vs baseline: 1.1220x; 1.1220x over previous
import functools

import jax
import jax.numpy as jnp
from jax import lax
from jax.experimental import pallas as pl
from jax.experimental.pallas import tpu as pltpu, tpu_sc as plsc

N = 10000
E = 320000
DF = 128
H = 64
T = 8
G = 128

NC = 2
NS = 16
NW = NC * NS
CH = 232
RB = 1
NBUF = 2
K = -(-E // (NW * CH * NBUF)) * NBUF
E_PAD = NW * K * CH
NPAD = -(-N // (NS * 8)) * NS * 8
RPW = NPAD // NS


def _bn_relu(h, g, b):
    m = jnp.mean(h, axis=0, keepdims=True)
    v = jnp.mean((h - m) * (h - m), axis=0, keepdims=True)
    return jax.nn.relu((h - m) / jnp.sqrt(v + 1e-5) * g + b)


def _onehot_t(batch_ref):
    rows = lax.broadcasted_iota(jnp.int32, (G, N), 0)
    return (rows == batch_ref[...]).astype(jnp.float32)


def _mlp_pool_body(bias_per_node, x_ref, batch_ref, w1, b1, g1, be1,
                   w2, b2, g2, be2, lw, lb, part_in, h_out, part_out):
    h = jnp.dot(x_ref[...], w1[...], preferred_element_type=jnp.float32)
    h = _bn_relu(h + b1[...], g1[...], be1[...])
    h = jnp.dot(h, w2[...], preferred_element_type=jnp.float32)
    h = _bn_relu(h + b2[...], g2[...], be2[...])
    h_out[...] = h
    oh = _onehot_t(batch_ref)
    pool = jnp.dot(oh, h, preferred_element_type=jnp.float32)
    if bias_per_node:
        bterm = jnp.sum(oh, axis=1, keepdims=True) * lb[...]
    else:
        bterm = lb[...]
    part_out[...] = (part_in[...]
                     + jnp.dot(pool, lw[...], preferred_element_type=jnp.float32)
                     + bterm)


def _gin_input_body(h_ref, a0_ref, a1_ref, eps_ref, out_ref):
    out_ref[...] = ((1.0 + eps_ref[0, 0]) * h_ref[...]
                    + a0_ref[0:N, :] + a1_ref[0:N, :])


_out_shapes = (jax.ShapeDtypeStruct((N, H), jnp.float32),
               jax.ShapeDtypeStruct((G, T), jnp.float32))
_mlp_pool0 = pl.pallas_call(
    functools.partial(_mlp_pool_body, True), out_shape=_out_shapes)
_mlp_pool = pl.pallas_call(
    functools.partial(_mlp_pool_body, False), out_shape=_out_shapes)

_gin_input = pl.pallas_call(
    _gin_input_body,
    out_shape=jax.ShapeDtypeStruct((N, H), jnp.float32),
)


def _sc_agg_body(h_hbm, srcs_hbm, dsts_hbm, out_hbm,
                 src_v, dst_v, rows, gsems, obuf, acc):
    cid = lax.axis_index("c")
    sid = lax.axis_index("s")
    wid = sid * NC + cid

    def _z(j, _):
        obuf[j // (H // 16), pl.ds((j % (H // 16)) * 16, 16)] = (
            jnp.zeros((16,), jnp.float32))
        return 0
    lax.fori_loop(0, RPW * (H // 16), _z, 0)
    pltpu.sync_copy(obuf, acc.at[pl.ds(sid * RPW, RPW)])
    plsc.subcore_barrier()

    pltpu.sync_copy(srcs_hbm.at[wid], src_v)
    pltpu.sync_copy(dsts_hbm.at[wid], dst_v)

    for b in range(NBUF):
        pltpu.async_copy(h_hbm.at[src_v.at[b]], rows[b], gsems[b])

    def _body(i, _):
        c = i * NBUF
        for b in range(NBUF):
            pltpu.make_async_copy(h_hbm.at[src_v.at[c + b]],
                                  rows[b], gsems[b]).wait()
            pltpu.sync_copy(rows[b], acc.at[dst_v.at[c + b]], add=True)

            @pl.when(i < K // NBUF - 1)
            def _():
                pltpu.async_copy(h_hbm.at[src_v.at[c + NBUF + b]],
                                 rows[b], gsems[b])
        return 0
    lax.fori_loop(0, K // NBUF, _body, 0)
    plsc.subcore_barrier()

    pltpu.sync_copy(acc.at[pl.ds(sid * RPW, RPW)], obuf)
    pltpu.sync_copy(obuf, out_hbm.at[cid, pl.ds(sid * RPW, RPW)])


@functools.lru_cache(maxsize=None)
def _make_sc_agg():
    return pl.kernel(
        _sc_agg_body,
        out_type=jax.ShapeDtypeStruct((NC, NPAD, H), jnp.float32),
        mesh=plsc.VectorSubcoreMesh(core_axis_name="c", subcore_axis_name="s",
                                    num_cores=NC, num_subcores=NS),
        scratch_types=[
            pltpu.VMEM((K, CH), jnp.int32),
            pltpu.VMEM((K, CH), jnp.int32),
            [pltpu.VMEM((CH, H), jnp.float32) for _ in range(NBUF)],
            [pltpu.SemaphoreType.DMA for _ in range(NBUF)],
            pltpu.VMEM((RPW, H), jnp.float32),
            pltpu.VMEM_SHARED((NPAD, H), jnp.float32),
        ],
        compiler_params=pltpu.CompilerParams(use_tc_tiling_on_sc=False),
    )


def _sc_agg(h, src, dst):
    return _make_sc_agg()(h, src, dst)


def kernel(x, edge_index, batch, fh_W1, fh_b1, fh_g1, fh_be1, fh_W2, fh_b2,
           fh_g2, fh_be2, c0_W1, c0_b1, c0_g1, c0_be1, c0_W2, c0_b2, c0_g2,
           c0_be2, c0_eps, c1_W1, c1_b1, c1_g1, c1_be1, c1_W2, c1_b2, c1_g2,
           c1_be2, c1_eps, lin0_W, lin0_b, lin1_W, lin1_b, lin2_W, lin2_b):
    r = lambda a: a.reshape(1, -1)
    batch_r = batch.reshape(1, N)

    src = jnp.concatenate(
        [edge_index[0], jnp.zeros((E_PAD - E,), jnp.int32)]).reshape(NW, K, CH)
    dst = jnp.concatenate(
        [edge_index[1], jnp.full((E_PAD - E,), N, jnp.int32)]).reshape(NW, K, CH)

    zpart = jnp.zeros((G, T), jnp.float32)

    h0, part0 = _mlp_pool0(x, batch_r, fh_W1, r(fh_b1), r(fh_g1), r(fh_be1),
                           fh_W2, r(fh_b2), r(fh_g2), r(fh_be2),
                           lin0_W, r(lin0_b), zpart)

    agg = _sc_agg(h0, src, dst)
    x1 = _gin_input(h0, agg[0], agg[1], c0_eps.reshape(1, 1))
    h1, part1 = _mlp_pool(x1, batch_r, c0_W1, r(c0_b1), r(c0_g1), r(c0_be1),
                          c0_W2, r(c0_b2), r(c0_g2), r(c0_be2),
                          lin1_W, r(lin1_b), part0)

    agg = _sc_agg(h1, src, dst)
    x2 = _gin_input(h1, agg[0], agg[1], c1_eps.reshape(1, 1))
    _, part2 = _mlp_pool(x2, batch_r, c1_W1, r(c1_b1), r(c1_g1), r(c1_be1),
                         c1_W2, r(c1_b2), r(c1_g2), r(c1_be2),
                         lin2_W, r(lin2_b), part1)
    return part2


# speedup vs baseline: 1.7961x; 1.6008x over previous
import functools

import jax
import jax.numpy as jnp
from jax import lax
from jax.experimental import pallas as pl
from jax.experimental.pallas import tpu as pltpu, tpu_sc as plsc

N = 10000
E = 320000
DF = 128
H = 64
T = 8
G = 128

NC = 2
NS = 16
NW = NC * NS
CH = 112
K = -(-E // (NW * CH * 2)) * 2
E_PAD = NW * K * CH
NPAD = -(-N // (NS * 8)) * NS * 8
RPW = NPAD // NS


def _bn_relu(h, g, b):
    m = jnp.mean(h, axis=0, keepdims=True)
    v = jnp.mean((h - m) * (h - m), axis=0, keepdims=True)
    return jax.nn.relu((h - m) / jnp.sqrt(v + 1e-5) * g + b)


def _onehot_t(batch_ref):
    rows = lax.broadcasted_iota(jnp.int32, (G, N), 0)
    return (rows == batch_ref[...]).astype(jnp.float32)


def _mlp_pool_body(bias_per_node, x_ref, batch_ref, w1, b1, g1, be1,
                   w2, b2, g2, be2, lw, lb, part_in, h_out, part_out):
    h = jnp.dot(x_ref[...], w1[...], preferred_element_type=jnp.float32)
    h = _bn_relu(h + b1[...], g1[...], be1[...])
    h = jnp.dot(h, w2[...], preferred_element_type=jnp.float32)
    h = _bn_relu(h + b2[...], g2[...], be2[...])
    h_out[0:N, :] = h
    h_out[N:NPAD, :] = jnp.zeros((NPAD - N, H), jnp.float32)
    oh = _onehot_t(batch_ref)
    pool = jnp.dot(oh, h, preferred_element_type=jnp.float32)
    if bias_per_node:
        bterm = jnp.sum(oh, axis=1, keepdims=True) * lb[...]
    else:
        bterm = lb[...]
    part_out[...] = (part_in[...]
                     + jnp.dot(pool, lw[...], preferred_element_type=jnp.float32)
                     + bterm)


def _gin_input_body(h_ref, a0_ref, a1_ref, eps_ref, out_ref):
    out_ref[...] = ((1.0 + eps_ref[0, 0]) * h_ref[0:N, :]
                    + a0_ref[0:N, :] + a1_ref[0:N, :])


_out_shapes = (jax.ShapeDtypeStruct((NPAD, H), jnp.float32),
               jax.ShapeDtypeStruct((G, T), jnp.float32))
_mlp_pool0 = pl.pallas_call(
    functools.partial(_mlp_pool_body, True), out_shape=_out_shapes)
_mlp_pool = pl.pallas_call(
    functools.partial(_mlp_pool_body, False), out_shape=_out_shapes)

_gin_input = pl.pallas_call(
    _gin_input_body,
    out_shape=jax.ShapeDtypeStruct((N, H), jnp.float32),
)


def _sc_agg_body(h_hbm, srcs_hbm, dsts_hbm, out_hbm,
                 sa0, sa1, da0, da1, rows, gsem, isem0, isem1, obuf,
                 acc, hsh):
    cid = lax.axis_index("c")
    sid = lax.axis_index("s")
    wid = sid * NC + cid
    sa = (sa0, sa1)
    da = (da0, da1)
    isem = (isem0, isem1)

    pltpu.async_copy(h_hbm.at[pl.ds(sid * RPW, RPW)],
                     hsh.at[pl.ds(sid * RPW, RPW)], gsem)

    def _z(j, _):
        obuf[j // (H // 16), pl.ds((j % (H // 16)) * 16, 16)] = (
            jnp.zeros((16,), jnp.float32))
        return 0
    lax.fori_loop(0, RPW * (H // 16), _z, 0)
    pltpu.sync_copy(obuf, acc.at[pl.ds(sid * RPW, RPW)])

    for b in range(2):
        pltpu.async_copy(srcs_hbm.at[wid, b], sa[b], isem[b])
        pltpu.async_copy(dsts_hbm.at[wid, b], da[b], isem[b])

    pltpu.make_async_copy(h_hbm.at[pl.ds(sid * RPW, RPW)],
                          hsh.at[pl.ds(sid * RPW, RPW)], gsem).wait()
    plsc.subcore_barrier()

    def _body(i, _):
        for b in range(2):
            j = i * 2 + b
            pltpu.make_async_copy(srcs_hbm.at[wid, j], sa[b], isem[b]).wait()
            pltpu.make_async_copy(dsts_hbm.at[wid, j], da[b], isem[b]).wait()
            pltpu.async_copy(hsh.at[sa[b]], rows, gsem).wait()
            pltpu.sync_copy(rows, acc.at[da[b]], add=True)

            @pl.when(j + 2 < K)
            def _():
                pltpu.async_copy(srcs_hbm.at[wid, j + 2], sa[b], isem[b])
                pltpu.async_copy(dsts_hbm.at[wid, j + 2], da[b], isem[b])
        return 0
    lax.fori_loop(0, K // 2, _body, 0)
    plsc.subcore_barrier()

    pltpu.sync_copy(acc.at[pl.ds(sid * RPW, RPW)],
                    out_hbm.at[cid, pl.ds(sid * RPW, RPW)])


@functools.lru_cache(maxsize=None)
def _make_sc_agg():
    return pl.kernel(
        _sc_agg_body,
        out_type=jax.ShapeDtypeStruct((NC, NPAD, H), jnp.float32),
        mesh=plsc.VectorSubcoreMesh(core_axis_name="c", subcore_axis_name="s",
                                    num_cores=NC, num_subcores=NS),
        scratch_types=[
            pltpu.VMEM((CH,), jnp.int32),
            pltpu.VMEM((CH,), jnp.int32),
            pltpu.VMEM((CH,), jnp.int32),
            pltpu.VMEM((CH,), jnp.int32),
            pltpu.VMEM((CH, H), jnp.float32),
            pltpu.SemaphoreType.DMA,
            pltpu.SemaphoreType.DMA,
            pltpu.SemaphoreType.DMA,
            pltpu.VMEM((RPW, H), jnp.float32),
            pltpu.VMEM_SHARED((NPAD, H), jnp.float32),
            pltpu.VMEM_SHARED((NPAD, H), jnp.float32),
        ],
        compiler_params=pltpu.CompilerParams(use_tc_tiling_on_sc=False),
    )


def _sc_agg(h, src, dst):
    return _make_sc_agg()(h, src, dst)


def kernel(x, edge_index, batch, fh_W1, fh_b1, fh_g1, fh_be1, fh_W2, fh_b2,
           fh_g2, fh_be2, c0_W1, c0_b1, c0_g1, c0_be1, c0_W2, c0_b2, c0_g2,
           c0_be2, c0_eps, c1_W1, c1_b1, c1_g1, c1_be1, c1_W2, c1_b2, c1_g2,
           c1_be2, c1_eps, lin0_W, lin0_b, lin1_W, lin1_b, lin2_W, lin2_b):
    r = lambda a: a.reshape(1, -1)
    batch_r = batch.reshape(1, N)

    src = jnp.concatenate(
        [edge_index[0], jnp.zeros((E_PAD - E,), jnp.int32)]).reshape(NW, K, CH)
    dst = jnp.concatenate(
        [edge_index[1], jnp.full((E_PAD - E,), N, jnp.int32)]).reshape(NW, K, CH)

    zpart = jnp.zeros((G, T), jnp.float32)

    h0, part0 = _mlp_pool0(x, batch_r, fh_W1, r(fh_b1), r(fh_g1), r(fh_be1),
                           fh_W2, r(fh_b2), r(fh_g2), r(fh_be2),
                           lin0_W, r(lin0_b), zpart)

    agg = _sc_agg(h0, src, dst)
    x1 = _gin_input(h0, agg[0], agg[1], c0_eps.reshape(1, 1))
    h1, part1 = _mlp_pool(x1, batch_r, c0_W1, r(c0_b1), r(c0_g1), r(c0_be1),
                          c0_W2, r(c0_b2), r(c0_g2), r(c0_be2),
                          lin1_W, r(lin1_b), part0)

    agg = _sc_agg(h1, src, dst)
    x2 = _gin_input(h1, agg[0], agg[1], c1_eps.reshape(1, 1))
    _, part2 = _mlp_pool(x2, batch_r, c1_W1, r(c1_b1), r(c1_g1), r(c1_be1),
                         c1_W2, r(c1_b2), r(c1_g2), r(c1_be2),
                         lin2_W, r(lin2_b), part1)
    return part2


# speedup vs baseline: 1.8678x; 1.0399x over previous
import functools

import jax
import jax.numpy as jnp
from jax import lax
from jax.experimental import pallas as pl
from jax.experimental.pallas import tpu as pltpu, tpu_sc as plsc

N = 10000
E = 320000
DF = 128
H = 64
T = 8
G = 128

NC = 2
NS = 16
NW = NC * NS
CH = 112
K = -(-E // (NW * CH * 2)) * 2
E_PAD = NW * K * CH
NPAD = -(-N // (NS * 8)) * NS * 8
RPW = NPAD // NS


def _bn_relu(h, g, b):
    m = jnp.mean(h, axis=0, keepdims=True)
    v = jnp.mean((h - m) * (h - m), axis=0, keepdims=True)
    return jax.nn.relu((h - m) / jnp.sqrt(v + 1e-5) * g + b)


def _onehot_t(batch_ref):
    rows = lax.broadcasted_iota(jnp.int32, (G, N), 0)
    return (rows == batch_ref[...]).astype(jnp.float32)


def _mlp_pool_body(bias_per_node, gin, *refs):
    if gin:
        (x_ref, a0_ref, a1_ref, eps_ref, batch_ref, w1, b1, g1, be1,
         w2, b2, g2, be2, lw, lb, part_in, h_out, part_out) = refs
        x = ((1.0 + eps_ref[0, 0]) * x_ref[0:N, :]
             + a0_ref[0:N, :] + a1_ref[0:N, :])
    else:
        (x_ref, batch_ref, w1, b1, g1, be1,
         w2, b2, g2, be2, lw, lb, part_in, h_out, part_out) = refs
        x = x_ref[...]
    h = jnp.dot(x, w1[...], preferred_element_type=jnp.float32)
    h = _bn_relu(h + b1[...], g1[...], be1[...])
    h = jnp.dot(h, w2[...], preferred_element_type=jnp.float32)
    h = _bn_relu(h + b2[...], g2[...], be2[...])
    h_out[0:N, :] = h
    h_out[N:NPAD, :] = jnp.zeros((NPAD - N, H), jnp.float32)
    oh = _onehot_t(batch_ref)
    pool = jnp.dot(oh, h, preferred_element_type=jnp.float32)
    if bias_per_node:
        bterm = jnp.sum(oh, axis=1, keepdims=True) * lb[...]
    else:
        bterm = lb[...]
    part_out[...] = (part_in[...]
                     + jnp.dot(pool, lw[...], preferred_element_type=jnp.float32)
                     + bterm)


_out_shapes = (jax.ShapeDtypeStruct((NPAD, H), jnp.float32),
               jax.ShapeDtypeStruct((G, T), jnp.float32))
_mlp_pool0 = pl.pallas_call(
    functools.partial(_mlp_pool_body, True, False), out_shape=_out_shapes)
_mlp_pool_gin = pl.pallas_call(
    functools.partial(_mlp_pool_body, False, True), out_shape=_out_shapes)


def _sc_agg_body(h_hbm, srcs_hbm, dsts_hbm, out_hbm,
                 sa0, sa1, da0, da1, rows, gsem, isem0, isem1, obuf,
                 acc, hsh):
    cid = lax.axis_index("c")
    sid = lax.axis_index("s")
    wid = sid * NC + cid
    sa = (sa0, sa1)
    da = (da0, da1)
    isem = (isem0, isem1)

    pltpu.async_copy(h_hbm.at[pl.ds(sid * RPW, RPW)],
                     hsh.at[pl.ds(sid * RPW, RPW)], gsem)

    def _z(j, _):
        obuf[j // (H // 16), pl.ds((j % (H // 16)) * 16, 16)] = (
            jnp.zeros((16,), jnp.float32))
        return 0
    lax.fori_loop(0, RPW * (H // 16), _z, 0)
    pltpu.sync_copy(obuf, acc.at[pl.ds(sid * RPW, RPW)])

    for b in range(2):
        pltpu.async_copy(srcs_hbm.at[wid, b], sa[b], isem[b])
        pltpu.async_copy(dsts_hbm.at[wid, b], da[b], isem[b])

    pltpu.make_async_copy(h_hbm.at[pl.ds(sid * RPW, RPW)],
                          hsh.at[pl.ds(sid * RPW, RPW)], gsem).wait()
    plsc.subcore_barrier()

    def _body(i, _):
        for b in range(2):
            j = i * 2 + b
            pltpu.make_async_copy(srcs_hbm.at[wid, j], sa[b], isem[b]).wait()
            pltpu.make_async_copy(dsts_hbm.at[wid, j], da[b], isem[b]).wait()
            pltpu.async_copy(hsh.at[sa[b]], rows, gsem).wait()
            pltpu.sync_copy(rows, acc.at[da[b]], add=True)

            @pl.when(j + 2 < K)
            def _():
                pltpu.async_copy(srcs_hbm.at[wid, j + 2], sa[b], isem[b])
                pltpu.async_copy(dsts_hbm.at[wid, j + 2], da[b], isem[b])
        return 0
    lax.fori_loop(0, K // 2, _body, 0)
    plsc.subcore_barrier()

    pltpu.sync_copy(acc.at[pl.ds(sid * RPW, RPW)],
                    out_hbm.at[cid, pl.ds(sid * RPW, RPW)])


@functools.lru_cache(maxsize=None)
def _make_sc_agg():
    return pl.kernel(
        _sc_agg_body,
        out_type=jax.ShapeDtypeStruct((NC, NPAD, H), jnp.float32),
        mesh=plsc.VectorSubcoreMesh(core_axis_name="c", subcore_axis_name="s",
                                    num_cores=NC, num_subcores=NS),
        scratch_types=[
            pltpu.VMEM((CH,), jnp.int32),
            pltpu.VMEM((CH,), jnp.int32),
            pltpu.VMEM((CH,), jnp.int32),
            pltpu.VMEM((CH,), jnp.int32),
            pltpu.VMEM((CH, H), jnp.float32),
            pltpu.SemaphoreType.DMA,
            pltpu.SemaphoreType.DMA,
            pltpu.SemaphoreType.DMA,
            pltpu.VMEM((RPW, H), jnp.float32),
            pltpu.VMEM_SHARED((NPAD, H), jnp.float32),
            pltpu.VMEM_SHARED((NPAD, H), jnp.float32),
        ],
        compiler_params=pltpu.CompilerParams(use_tc_tiling_on_sc=False),
    )


def _sc_agg(h, src, dst):
    return _make_sc_agg()(h, src, dst)


def kernel(x, edge_index, batch, fh_W1, fh_b1, fh_g1, fh_be1, fh_W2, fh_b2,
           fh_g2, fh_be2, c0_W1, c0_b1, c0_g1, c0_be1, c0_W2, c0_b2, c0_g2,
           c0_be2, c0_eps, c1_W1, c1_b1, c1_g1, c1_be1, c1_W2, c1_b2, c1_g2,
           c1_be2, c1_eps, lin0_W, lin0_b, lin1_W, lin1_b, lin2_W, lin2_b):
    r = lambda a: a.reshape(1, -1)
    batch_r = batch.reshape(1, N)

    src = jnp.concatenate(
        [edge_index[0], jnp.zeros((E_PAD - E,), jnp.int32)]).reshape(NW, K, CH)
    dst = jnp.concatenate(
        [edge_index[1], jnp.full((E_PAD - E,), N, jnp.int32)]).reshape(NW, K, CH)

    zpart = jnp.zeros((G, T), jnp.float32)

    h0, part0 = _mlp_pool0(x, batch_r, fh_W1, r(fh_b1), r(fh_g1), r(fh_be1),
                           fh_W2, r(fh_b2), r(fh_g2), r(fh_be2),
                           lin0_W, r(lin0_b), zpart)

    agg = _sc_agg(h0, src, dst)
    h1, part1 = _mlp_pool_gin(h0, agg[0], agg[1], c0_eps.reshape(1, 1),
                              batch_r, c0_W1, r(c0_b1), r(c0_g1), r(c0_be1),
                              c0_W2, r(c0_b2), r(c0_g2), r(c0_be2),
                              lin1_W, r(lin1_b), part0)

    agg = _sc_agg(h1, src, dst)
    _, part2 = _mlp_pool_gin(h1, agg[0], agg[1], c1_eps.reshape(1, 1),
                             batch_r, c1_W1, r(c1_b1), r(c1_g1), r(c1_be1),
                             c1_W2, r(c1_b2), r(c1_g2), r(c1_be2),
                             lin2_W, r(lin2_b), part1)
    return part2
